# Initial kernel scaffold; baseline (speedup 1.0000x reference)
import jax
import jax.numpy as jnp
from jax.experimental import pallas as pl


def kernel(x, edge_index, edge_attr, W1, b1, W2, b2, W3, b3):
    raise NotImplementedError("write your pallas kernel here")


# speedup vs baseline: 30.0191x; 30.0191x over previous
import functools

import jax
import jax.numpy as jnp
from jax import lax
from jax.experimental import pallas as pl
from jax.experimental.pallas import tpu as pltpu
from jax.experimental.pallas import tpu_sc as plsc

N = 10000
NP = 10240
E = 320000
F = 16
G = 128
NT = 32
GP = (E + NT * G - 1) // (NT * G)
EP = NT * G * GP
ROWS_PER_TILE = NP // 16

_MESH = dict(core_axis_name="c", subcore_axis_name="s")


@functools.partial(
    pl.kernel,
    mesh=plsc.VectorSubcoreMesh(**_MESH),
    out_type=jax.ShapeDtypeStruct((2, NP), jnp.float32),
    scratch_types=[
        pltpu.VMEM_SHARED((NP,), jnp.float32),
        pltpu.VMEM((GP, G), jnp.int32),
        pltpu.VMEM((GP, G), jnp.float32),
    ],
)
def _sc_degree(colp, ewp, zn, deg_out, acc_s, col_v, ew_v):
    cid = lax.axis_index("c")
    sid = lax.axis_index("s")
    g = cid * 16 + sid

    @pl.when(sid == 0)
    def _init():
        pltpu.sync_copy(zn, acc_s)

    plsc.subcore_barrier()

    pltpu.sync_copy(colp.at[g], col_v)
    pltpu.sync_copy(ewp.at[g], ew_v)

    def body(j, carry):
        pltpu.sync_copy(ew_v.at[j], acc_s.at[col_v.at[j]], add=True)
        return carry

    lax.fori_loop(0, GP, body, 0)
    plsc.subcore_barrier()

    @pl.when(sid == 0)
    def _readout():
        pltpu.sync_copy(acc_s, deg_out.at[cid])


@functools.partial(
    pl.kernel,
    mesh=plsc.VectorSubcoreMesh(**_MESH),
    compiler_params=pltpu.CompilerParams(use_tc_tiling_on_sc=False),
    out_type=jax.ShapeDtypeStruct((2, 16, ROWS_PER_TILE, F), jnp.float32),
    scratch_types=[
        pltpu.VMEM_SHARED((NP, F), jnp.float32),
        pltpu.VMEM((GP, G), jnp.int32),
        pltpu.VMEM((GP, G), jnp.int32),
        pltpu.VMEM((GP, G), jnp.float32),
        pltpu.VMEM((G, F), jnp.float32),
        pltpu.SemaphoreType.DMA,
    ],
)
def _sc_aggregate(rowp, colp, ewp, y_hbm, znf, acc_out,
                  acc_s, row_v, col_v, ew_v, msg_v, sem):
    cid = lax.axis_index("c")
    sid = lax.axis_index("s")
    g = cid * 16 + sid
    ns = pl.ds(pl.multiple_of(sid * ROWS_PER_TILE, 8), ROWS_PER_TILE)

    pltpu.sync_copy(znf.at[sid], acc_s.at[ns])
    plsc.subcore_barrier()

    pltpu.sync_copy(rowp.at[g], row_v)
    pltpu.sync_copy(colp.at[g], col_v)
    pltpu.sync_copy(ewp.at[g], ew_v)

    def group(j, carry):
        pltpu.async_copy(y_hbm.at[row_v.at[j]], msg_v, sem).wait()

        def scale(sg, c2):
            e0 = sg * 16
            ew16 = ew_v[j, pl.ds(e0, 16)]
            for k in range(16):
                msg_v[e0 + k, :] = msg_v[e0 + k, :] * ew16[k]
            return c2

        lax.fori_loop(0, G // 16, scale, 0)
        pltpu.sync_copy(msg_v, acc_s.at[col_v.at[j]], add=True)
        return carry

    lax.fori_loop(0, GP, group, 0)
    plsc.subcore_barrier()

    pltpu.sync_copy(acc_s.at[ns], acc_out.at[cid, sid])


def _mm_body(x_ref, w_ref, o_ref):
    o_ref[...] = jnp.dot(x_ref[...], w_ref[...],
                         preferred_element_type=jnp.float32)


def _tc_matmul(x, w):
    return pl.pallas_call(
        _mm_body,
        out_shape=jax.ShapeDtypeStruct((x.shape[0], w.shape[1]), jnp.float32),
    )(x, w)


def _norm_body(dega_ref, degb_ref, xw_ref, dis_ref, y_ref, self_ref):
    deg = dega_ref[...] + degb_ref[...] + 1.0
    dis = 1.0 / jnp.sqrt(deg)
    y = dis * xw_ref[...]
    dis_ref[...] = dis
    y_ref[...] = y
    self_ref[...] = dis * y


def _tc_norm(dega, degb, xw):
    return pl.pallas_call(
        _norm_body,
        out_shape=[
            jax.ShapeDtypeStruct((N, 1), jnp.float32),
            jax.ShapeDtypeStruct((N, F), jnp.float32),
            jax.ShapeDtypeStruct((N, F), jnp.float32),
        ],
    )(dega, degb, xw)


def _mid_body(acca_ref, accb_ref, self_ref, dis_ref, b_ref, w_ref,
              y2_ref, self2_ref):
    h = dis_ref[...] * (acca_ref[...] + accb_ref[...]) \
        + self_ref[...] + b_ref[...]
    h = jnp.maximum(h, 0.0)
    xw2 = jnp.dot(h, w_ref[...], preferred_element_type=jnp.float32)
    y2 = dis_ref[...] * xw2
    y2_ref[...] = y2
    self2_ref[...] = dis_ref[...] * y2


def _tc_mid(acca, accb, self1, dis, b1, w2p):
    return pl.pallas_call(
        _mid_body,
        out_shape=[
            jax.ShapeDtypeStruct((N, F), jnp.float32),
            jax.ShapeDtypeStruct((N, F), jnp.float32),
        ],
    )(acca, accb, self1, dis, b1, w2p)


def _post_body(acca_ref, accb_ref, self2_ref, dis_ref, b2_ref, w3_ref,
               b3_ref, out_ref, x2_ref):
    x2 = dis_ref[...] * (acca_ref[...] + accb_ref[...]) \
        + self2_ref[...] + b2_ref[...]
    r = jnp.maximum(x2, 0.0)
    z = jnp.sum(r * w3_ref[...], axis=1, keepdims=True) + b3_ref[...]
    out_ref[...] = jax.nn.sigmoid(z)
    x2_ref[...] = x2


def _tc_post(acca, accb, self2, dis, b2p, w3row, b3):
    return pl.pallas_call(
        _post_body,
        out_shape=[
            jax.ShapeDtypeStruct((N, 1), jnp.float32),
            jax.ShapeDtypeStruct((N, F), jnp.float32),
        ],
    )(acca, accb, self2, dis, b2p, w3row, b3)


def _pad_edges(v, fill):
    return jnp.concatenate(
        [v, jnp.full((EP - E,), fill, v.dtype)]).reshape(NT, GP, G)


def kernel(x, edge_index, edge_attr, W1, b1, W2, b2, W3, b3):
    row = edge_index[0].astype(jnp.int32)
    col = edge_index[1].astype(jnp.int32)
    rowp = _pad_edges(row, 0)
    colp = _pad_edges(col, 0)
    ewp = _pad_edges(edge_attr, 0.0)

    zn = jnp.zeros((NP,), jnp.float32)
    znf = jnp.zeros((16, ROWS_PER_TILE, F), jnp.float32)

    w2p = jnp.zeros((F, F), jnp.float32).at[:, :2].set(W2)
    b2p = jnp.zeros((1, F), jnp.float32).at[0, :2].set(b2)
    w3row = jnp.zeros((1, F), jnp.float32).at[0, :2].set(W3[:, 0])
    b1r = b1.reshape(1, F)
    b3r = b3.reshape(1, 1)

    xw1 = _tc_matmul(x, W1)

    degp = _sc_degree(colp, ewp, zn)
    dega = degp[0, :N].reshape(N, 1)
    degb = degp[1, :N].reshape(N, 1)

    dis, y1, self1 = _tc_norm(dega, degb, xw1)

    y1p = jnp.pad(y1, ((0, NP - N), (0, 0)))
    acc1 = _sc_aggregate(rowp, colp, ewp, y1p, znf)
    acc1 = acc1.reshape(2, NP, F)
    y2, self2 = _tc_mid(acc1[0, :N], acc1[1, :N], self1, dis, b1r, w2p)

    y2p = jnp.pad(y2, ((0, NP - N), (0, 0)))
    acc2 = _sc_aggregate(rowp, colp, ewp, y2p, znf)
    acc2 = acc2.reshape(2, NP, F)
    out, x2p = _tc_post(acc2[0, :N], acc2[1, :N], self2, dis, b2p, w3row, b3r)

    return (out, x2p[:, :2])
